# Initial kernel scaffold; baseline (speedup 1.0000x reference)
import functools

import jax
import jax.numpy as jnp
from jax import lax
from jax.experimental import pallas as pl
from jax.experimental.pallas import tpu as pltpu
from jax.experimental.pallas import tpu_sc as plsc

N = 10000
E = 320000
D = 128
NP = 10240
NC, NS = 2, 16
NT = NC * NS
EC = E // NT
B = 125
CH = EC // B
DEN_R, DEN_C = 48, 512
RT = DEN_R // NS
ROWS_PT = NP // NS
CPB = 80

_mesh = plsc.VectorSubcoreMesh(
    core_axis_name="c", subcore_axis_name="s", num_cores=NC, num_subcores=NS)


def _proj_body(x_ref, w_ref, att_ref, h_ref, as_ref, ad_ref):
    for r in range(2):
        h = jnp.dot(x_ref[...], w_ref[r], preferred_element_type=jnp.float32)
        h_ref[r] = h
        as_ref[r] = jnp.sum(h * att_ref[r, 0][None, :], axis=1)
        ad_ref[r] = jnp.sum(h * att_ref[r, 1][None, :], axis=1)


_proj = pl.pallas_call(
    _proj_body,
    out_shape=[
        jax.ShapeDtypeStruct((2, NP, D), jnp.float32),
        jax.ShapeDtypeStruct((2, NP), jnp.float32),
        jax.ShapeDtypeStruct((2, NP), jnp.float32),
    ],
    interpret=False,
)


def _edge_body(src_h, dst_h, tt_h, asrc_h, adst_h,
               w_h, gi_h, di_h, den_h,
               src_v, dst_v, tt_v, as_v, ad_v, w_v, gi_v, di_v,
               den_v, rowi_v, den_sh):
    cid = lax.axis_index("c")
    sid = lax.axis_index("s")
    wid = cid * NS + sid
    base = wid * EC

    pltpu.sync_copy(src_h.at[pl.ds(base, EC)], src_v)
    pltpu.sync_copy(dst_h.at[pl.ds(base, EC)], dst_v)
    pltpu.sync_copy(tt_h.at[pl.ds(base, EC)], tt_v)
    pltpu.sync_copy(asrc_h, as_v)
    pltpu.sync_copy(adst_h, ad_v)

    zeros16 = jnp.zeros((16,), jnp.float32)

    def zero_body(i, carry):
        r = i // (DEN_C // 16)
        c = (i % (DEN_C // 16)) * 16
        den_v[r, pl.ds(c, 16)] = zeros16
        return carry

    lax.fori_loop(0, DEN_R * DEN_C // 16, zero_body, 0)

    iota16 = lax.iota(jnp.int32, 16)
    for k in range(DEN_R // 16):
        rowi_v[pl.ds(k * 16, 16)] = iota16 + k * 16

    def edge_body(i, carry):
        off = i * 16
        s = src_v[pl.ds(off, 16)]
        d = dst_v[pl.ds(off, 16)]
        t = tt_v[pl.ds(off, 16)]
        a1 = plsc.load_gather(as_v, [t, s])
        a2 = plsc.load_gather(ad_v, [t, d])
        e = a1 + a2
        e = jnp.where(e >= 0.0, e, e * jnp.float32(0.2))
        wv = jnp.exp(e)
        di = t * NP + d
        w_v[pl.ds(off, 16)] = wv
        gi_v[pl.ds(off, 16)] = t * NP + s
        di_v[pl.ds(off, 16)] = di
        plsc.addupdate_scatter(
            den_v, [jnp.right_shift(di, 9), jnp.bitwise_and(di, 511)], wv)
        return carry

    lax.fori_loop(0, EC // 16, edge_body, 0)

    pltpu.sync_copy(w_v, w_h.at[pl.ds(base, EC)])
    pltpu.sync_copy(gi_v, gi_h.at[pl.ds(base, EC)])
    pltpu.sync_copy(di_v, di_h.at[pl.ds(base, EC)])

    @pl.when(sid == 0)
    def _():
        pltpu.sync_copy(den_v, den_sh)

    plsc.subcore_barrier()

    @pl.when(sid != 0)
    def _():
        pltpu.sync_copy(den_v, den_sh.at[rowi_v], add=True)

    plsc.subcore_barrier()

    @pl.when(sid == 0)
    def _():
        pltpu.sync_copy(den_sh, den_v)
        pltpu.sync_copy(den_v, den_h.at[cid])


_edge = functools.partial(
    pl.kernel,
    out_type=[
        jax.ShapeDtypeStruct((E,), jnp.float32),
        jax.ShapeDtypeStruct((E,), jnp.int32),
        jax.ShapeDtypeStruct((E,), jnp.int32),
        jax.ShapeDtypeStruct((NC, DEN_R, DEN_C), jnp.float32),
    ],
    mesh=_mesh,
    scratch_types=[
        pltpu.VMEM((EC,), jnp.int32),
        pltpu.VMEM((EC,), jnp.int32),
        pltpu.VMEM((EC,), jnp.int32),
        pltpu.VMEM((2, NP), jnp.float32),
        pltpu.VMEM((2, NP), jnp.float32),
        pltpu.VMEM((EC,), jnp.float32),
        pltpu.VMEM((EC,), jnp.int32),
        pltpu.VMEM((EC,), jnp.int32),
        pltpu.VMEM((DEN_R, DEN_C), jnp.float32),
        pltpu.VMEM((DEN_R,), jnp.int32),
        pltpu.VMEM_SHARED((DEN_R, DEN_C), jnp.float32),
    ],
    interpret=False,
)(_edge_body)


def _agg_body(hcat_h, gi2_h, dst2_h, di_h, w_h, den_h,
              part_h,
              p0_v, p1_v, rden_v, gi2_v, dst2_v, di_v, w_v, al_v, rows_v,
              acc_sh, rden_sh, sem):
    cid = lax.axis_index("c")
    sid = lax.axis_index("s")
    wid = cid * NS + sid

    pltpu.sync_copy(den_h.at[0, pl.ds(sid * RT, RT)], p0_v)
    pltpu.sync_copy(den_h.at[1, pl.ds(sid * RT, RT)], p1_v)

    def rden_body(i, carry):
        r = i // (DEN_C // 16)
        c = (i % (DEN_C // 16)) * 16
        v0 = p0_v[r, pl.ds(c, 16)]
        v1 = p1_v[r, pl.ds(c, 16)]
        p0_v[r, pl.ds(c, 16)] = 1.0 / (v0 + v1 + jnp.float32(1e-16))
        return carry

    lax.fori_loop(0, RT * (DEN_C // 16), rden_body, 0)
    pltpu.sync_copy(p0_v, rden_sh.at[pl.ds(sid * RT, RT)])

    zeros16 = jnp.zeros((16,), jnp.float32)

    def zrow(i, carry):
        r = i // (D // 16)
        c = (i % (D // 16)) * 16
        rows_v[r, pl.ds(c, 16)] = zeros16
        return carry

    lax.fori_loop(0, CPB * (D // 16), zrow, 0)
    for k in range(ROWS_PT // CPB):
        pltpu.sync_copy(rows_v.at[pl.ds(0, CPB)],
                        acc_sh.at[pl.ds(sid * ROWS_PT + k * CPB, CPB)])
    plsc.subcore_barrier()
    pltpu.sync_copy(rden_sh, rden_v)

    pltpu.sync_copy(gi2_h.at[pl.ds(wid * CH, CH)], gi2_v)
    pltpu.sync_copy(dst2_h.at[pl.ds(wid * CH, CH)], dst2_v)
    pltpu.sync_copy(di_h.at[pl.ds(wid * EC, EC)], di_v)
    pltpu.sync_copy(w_h.at[pl.ds(wid * EC, EC)], w_v)

    def al_body(i, carry):
        off = i * 16
        di = di_v[pl.ds(off, 16)]
        rd = plsc.load_gather(
            rden_v, [jnp.right_shift(di, 9), jnp.bitwise_and(di, 511)])
        al_v[pl.ds(off, 16)] = w_v[pl.ds(off, 16)] * rd
        return carry

    lax.fori_loop(0, EC // 16, al_body, 0)

    def chunk_body(j, carry):
        pltpu.async_copy(hcat_h.at[gi2_v.at[j]], rows_v, sem).wait()
        cb = j * B

        def edge_mul(q, c2):
            aspl = plsc.load_gather(al_v, [jnp.full((16,), cb + q, jnp.int32)])
            for c in range(D // 16):
                rows_v[q, pl.ds(c * 16, 16)] = (
                    rows_v[q, pl.ds(c * 16, 16)] * aspl)
            return c2

        lax.fori_loop(0, B, edge_mul, 0)
        pltpu.sync_copy(rows_v, acc_sh.at[dst2_v.at[j]], add=True)
        return carry

    lax.fori_loop(0, CH, chunk_body, 0)

    plsc.subcore_barrier()
    for k in range(ROWS_PT // CPB):
        pltpu.sync_copy(acc_sh.at[pl.ds(sid * ROWS_PT + k * CPB, CPB)],
                        rows_v.at[pl.ds(0, CPB)])
        pltpu.sync_copy(rows_v.at[pl.ds(0, CPB)],
                        part_h.at[cid, pl.ds(sid * ROWS_PT + k * CPB, CPB)])


_agg = functools.partial(
    pl.kernel,
    out_type=jax.ShapeDtypeStruct((NC, NP, D), jnp.float32),
    mesh=_mesh,
    scratch_types=[
        pltpu.VMEM((RT, DEN_C), jnp.float32),
        pltpu.VMEM((RT, DEN_C), jnp.float32),
        pltpu.VMEM((DEN_R, DEN_C), jnp.float32),
        pltpu.VMEM((CH, B), jnp.int32),
        pltpu.VMEM((CH, B), jnp.int32),
        pltpu.VMEM((EC,), jnp.int32),
        pltpu.VMEM((EC,), jnp.float32),
        pltpu.VMEM((EC,), jnp.float32),
        pltpu.VMEM((B, D), jnp.float32),
        pltpu.VMEM_SHARED((NP, D), jnp.float32),
        pltpu.VMEM_SHARED((DEN_R, DEN_C), jnp.float32),
        pltpu.SemaphoreType.DMA,
    ],
    interpret=False,
)(_agg_body)


def _fin_body(p_ref, b_ref, o_ref):
    o_ref[...] = p_ref[0] + p_ref[1] + b_ref[...]


_fin = pl.pallas_call(
    _fin_body,
    out_shape=jax.ShapeDtypeStruct((NP, D), jnp.float32),
    interpret=False,
)


def kernel(x, edge_index, edge_type, W_login, a_src_login, a_dst_login,
           b_login, W_exec, a_src_exec, a_dst_exec, b_exec):
    x_pad = jnp.pad(x, ((0, NP - N), (0, 0)))
    Ws = jnp.stack([W_login, W_exec])
    atts = jnp.stack([jnp.stack([a_src_login, a_dst_login]),
                      jnp.stack([a_src_exec, a_dst_exec])])
    src = edge_index[0]
    dst = edge_index[1]

    h_cat, asrc, adst = _proj(x_pad, Ws, atts)
    w, gi, di, den = _edge(src, dst, edge_type, asrc, adst)
    part = _agg(h_cat.reshape(2 * NP, D), gi.reshape(E // B, B),
                dst.reshape(E // B, B), di, w, den)
    out = _fin(part, (b_login + b_exec).reshape(1, D))
    return out[:N]


# speedup vs baseline: 47.1206x; 47.1206x over previous
import functools

import jax
import jax.numpy as jnp
from jax import lax
from jax.experimental import pallas as pl
from jax.experimental.pallas import tpu as pltpu
from jax.experimental.pallas import tpu_sc as plsc

N = 10000
E = 320000
D = 128
NP = 10240
NC, NS = 2, 16
NT = NC * NS
EC = E // NT
EB = 2000
NB = EC // EB
B = 125
CH = EC // B
DEN = 2 * NP
DST = DEN // NS
ROWS_PT = NP // NS
CPB = 80

_mesh = plsc.VectorSubcoreMesh(
    core_axis_name="c", subcore_axis_name="s", num_cores=NC, num_subcores=NS)
_sc_params = pltpu.CompilerParams(needs_layout_passes=False)


def _proj_body(x_ref, w_ref, att_ref, h_ref, as_ref, ad_ref):
    for r in range(2):
        h = jnp.dot(x_ref[...], w_ref[r], preferred_element_type=jnp.float32)
        h_ref[r] = h
        as_ref[r] = jnp.sum(h * att_ref[r, 0][None, :], axis=1)
        ad_ref[r] = jnp.sum(h * att_ref[r, 1][None, :], axis=1)


_proj = pl.pallas_call(
    _proj_body,
    out_shape=[
        jax.ShapeDtypeStruct((2, NP, D), jnp.float32),
        jax.ShapeDtypeStruct((2, NP), jnp.float32),
        jax.ShapeDtypeStruct((2, NP), jnp.float32),
    ],
    interpret=False,
)


def _edge_body(src_h, dst_h, tt_h, asrc_h, adst_h,
               w_h, gi_h, di_h, den_h,
               src_v, dst_v, tt_v, as_v, ad_v, w_v, gi_v, di_v,
               den_v, acc_v, tmp_v, den_sh):
    cid = lax.axis_index("c")
    sid = lax.axis_index("s")
    wid = cid * NS + sid
    base = wid * EC

    pltpu.sync_copy(asrc_h, as_v)
    pltpu.sync_copy(adst_h, ad_v)

    zeros16 = jnp.zeros((16,), jnp.float32)

    def zero_body(i, carry):
        den_v[pl.ds(i * 16, 16)] = zeros16
        return carry

    lax.fori_loop(0, DEN // 16, zero_body, 0)

    def block_body(b, carry):
        boff = base + b * EB
        pltpu.sync_copy(src_h.at[pl.ds(boff, EB)], src_v)
        pltpu.sync_copy(dst_h.at[pl.ds(boff, EB)], dst_v)
        pltpu.sync_copy(tt_h.at[pl.ds(boff, EB)], tt_v)

        def edge_body(i, c2):
            off = i * 16
            s = src_v[pl.ds(off, 16)]
            d = dst_v[pl.ds(off, 16)]
            t = tt_v[pl.ds(off, 16)]
            gi = t * NP + s
            di = t * NP + d
            a1 = plsc.load_gather(as_v, [gi])
            a2 = plsc.load_gather(ad_v, [di])
            e = a1 + a2
            e = jnp.where(e >= 0.0, e, e * jnp.float32(0.2))
            wv = jnp.exp(e)
            w_v[pl.ds(off, 16)] = wv
            gi_v[pl.ds(off, 16)] = gi
            di_v[pl.ds(off, 16)] = di
            plsc.addupdate_scatter(den_v, [di], wv)
            return c2

        lax.fori_loop(0, EB // 16, edge_body, 0)
        pltpu.sync_copy(w_v, w_h.at[pl.ds(boff, EB)])
        pltpu.sync_copy(gi_v, gi_h.at[pl.ds(boff, EB)])
        pltpu.sync_copy(di_v, di_h.at[pl.ds(boff, EB)])
        return carry

    lax.fori_loop(0, NB, block_body, 0)

    pltpu.sync_copy(den_v, den_sh.at[sid])
    plsc.subcore_barrier()

    stripe = sid * DST
    pltpu.sync_copy(den_sh.at[0, pl.ds(stripe, DST)], acc_v)
    for k in range(1, NS):
        pltpu.sync_copy(den_sh.at[k, pl.ds(stripe, DST)], tmp_v)

        def add_body(i, carry):
            o = i * 16
            acc_v[pl.ds(o, 16)] = acc_v[pl.ds(o, 16)] + tmp_v[pl.ds(o, 16)]
            return carry

        lax.fori_loop(0, DST // 16, add_body, 0)

    pltpu.sync_copy(acc_v, den_h.at[cid, pl.ds(stripe, DST)])


_edge = functools.partial(
    pl.kernel,
    out_type=[
        jax.ShapeDtypeStruct((E,), jnp.float32),
        jax.ShapeDtypeStruct((E,), jnp.int32),
        jax.ShapeDtypeStruct((E,), jnp.int32),
        jax.ShapeDtypeStruct((NC, DEN), jnp.float32),
    ],
    mesh=_mesh,
    scratch_types=[
        pltpu.VMEM((EB,), jnp.int32),
        pltpu.VMEM((EB,), jnp.int32),
        pltpu.VMEM((EB,), jnp.int32),
        pltpu.VMEM((DEN,), jnp.float32),
        pltpu.VMEM((DEN,), jnp.float32),
        pltpu.VMEM((EB,), jnp.float32),
        pltpu.VMEM((EB,), jnp.int32),
        pltpu.VMEM((EB,), jnp.int32),
        pltpu.VMEM((DEN,), jnp.float32),
        pltpu.VMEM((DST,), jnp.float32),
        pltpu.VMEM((DST,), jnp.float32),
        pltpu.VMEM_SHARED((NS, DEN), jnp.float32),
    ],
    compiler_params=_sc_params,
    interpret=False,
)(_edge_body)


def _alpha_body(w_h, di_h, den_h,
                al_h,
                p0_v, p1_v, rden_v, w_v, di_v,
                rden_sh):
    cid = lax.axis_index("c")
    sid = lax.axis_index("s")
    wid = cid * NS + sid
    base = wid * EC
    stripe = sid * DST

    pltpu.sync_copy(den_h.at[0, pl.ds(stripe, DST)], p0_v)
    pltpu.sync_copy(den_h.at[1, pl.ds(stripe, DST)], p1_v)

    def rden_body(i, carry):
        o = i * 16
        v0 = p0_v[pl.ds(o, 16)]
        v1 = p1_v[pl.ds(o, 16)]
        p0_v[pl.ds(o, 16)] = 1.0 / (v0 + v1 + jnp.float32(1e-16))
        return carry

    lax.fori_loop(0, DST // 16, rden_body, 0)
    pltpu.sync_copy(p0_v, rden_sh.at[pl.ds(stripe, DST)])
    plsc.subcore_barrier()
    pltpu.sync_copy(rden_sh, rden_v)

    def block_body(b, carry):
        boff = base + b * EB
        pltpu.sync_copy(w_h.at[pl.ds(boff, EB)], w_v)
        pltpu.sync_copy(di_h.at[pl.ds(boff, EB)], di_v)

        def al_body(i, c2):
            off = i * 16
            di = di_v[pl.ds(off, 16)]
            rd = plsc.load_gather(rden_v, [di])
            w_v[pl.ds(off, 16)] = w_v[pl.ds(off, 16)] * rd
            return c2

        lax.fori_loop(0, EB // 16, al_body, 0)
        pltpu.sync_copy(w_v, al_h.at[pl.ds(boff, EB)])
        return carry

    lax.fori_loop(0, NB, block_body, 0)


_alpha = functools.partial(
    pl.kernel,
    out_type=jax.ShapeDtypeStruct((E,), jnp.float32),
    mesh=_mesh,
    scratch_types=[
        pltpu.VMEM((DST,), jnp.float32),
        pltpu.VMEM((DST,), jnp.float32),
        pltpu.VMEM((DEN,), jnp.float32),
        pltpu.VMEM((EB,), jnp.float32),
        pltpu.VMEM((EB,), jnp.int32),
        pltpu.VMEM_SHARED((DEN,), jnp.float32),
    ],
    compiler_params=_sc_params,
    interpret=False,
)(_alpha_body)


def _agg_body(hcat_h, gi2_h, dst2_h, al_h,
              part_h,
              gi2_v, dst2_v, al_v, rows_v,
              acc_sh, sem):
    cid = lax.axis_index("c")
    sid = lax.axis_index("s")
    wid = cid * NS + sid

    zeros16 = jnp.zeros((16,), jnp.float32)

    def zrow(i, carry):
        r = i // (D // 16)
        c = (i % (D // 16)) * 16
        rows_v[r, pl.ds(c, 16)] = zeros16
        return carry

    lax.fori_loop(0, CPB * (D // 16), zrow, 0)
    for k in range(ROWS_PT // CPB):
        pltpu.sync_copy(rows_v.at[pl.ds(0, CPB)],
                        acc_sh.at[pl.ds(sid * ROWS_PT + k * CPB, CPB)])
    plsc.subcore_barrier()

    pltpu.sync_copy(gi2_h.at[pl.ds(wid * CH, CH)], gi2_v)
    pltpu.sync_copy(dst2_h.at[pl.ds(wid * CH, CH)], dst2_v)
    pltpu.sync_copy(al_h.at[pl.ds(wid * EC, EC)], al_v)

    def chunk_body(j, carry):
        pltpu.async_copy(hcat_h.at[gi2_v.at[j]], rows_v, sem).wait()
        cb = j * B

        def edge_mul(q, c2):
            aspl = plsc.load_gather(al_v, [jnp.full((16,), cb + q, jnp.int32)])
            for c in range(D // 16):
                rows_v[q, pl.ds(c * 16, 16)] = (
                    rows_v[q, pl.ds(c * 16, 16)] * aspl)
            return c2

        lax.fori_loop(0, B, edge_mul, 0)
        pltpu.sync_copy(rows_v, acc_sh.at[dst2_v.at[j]], add=True)
        return carry

    lax.fori_loop(0, CH, chunk_body, 0)

    plsc.subcore_barrier()
    for k in range(ROWS_PT // CPB):
        pltpu.sync_copy(acc_sh.at[pl.ds(sid * ROWS_PT + k * CPB, CPB)],
                        rows_v.at[pl.ds(0, CPB)])
        pltpu.sync_copy(rows_v.at[pl.ds(0, CPB)],
                        part_h.at[cid, pl.ds(sid * ROWS_PT + k * CPB, CPB)])


_agg = functools.partial(
    pl.kernel,
    out_type=jax.ShapeDtypeStruct((NC, NP, D), jnp.float32),
    mesh=_mesh,
    scratch_types=[
        pltpu.VMEM((CH, B), jnp.int32),
        pltpu.VMEM((CH, B), jnp.int32),
        pltpu.VMEM((EC,), jnp.float32),
        pltpu.VMEM((B, D), jnp.float32),
        pltpu.VMEM_SHARED((NP, D), jnp.float32),
        pltpu.SemaphoreType.DMA,
    ],
    compiler_params=_sc_params,
    interpret=False,
)(_agg_body)


def _fin_body(p_ref, b_ref, o_ref):
    o_ref[...] = p_ref[0] + p_ref[1] + b_ref[...]


_fin = pl.pallas_call(
    _fin_body,
    out_shape=jax.ShapeDtypeStruct((NP, D), jnp.float32),
    interpret=False,
)


def kernel(x, edge_index, edge_type, W_login, a_src_login, a_dst_login,
           b_login, W_exec, a_src_exec, a_dst_exec, b_exec):
    x_pad = jnp.pad(x, ((0, NP - N), (0, 0)))
    Ws = jnp.stack([W_login, W_exec])
    atts = jnp.stack([jnp.stack([a_src_login, a_dst_login]),
                      jnp.stack([a_src_exec, a_dst_exec])])
    src = edge_index[0]
    dst = edge_index[1]

    h_cat, asrc, adst = _proj(x_pad, Ws, atts)
    w, gi, di, den = _edge(src, dst, edge_type,
                           asrc.reshape(2 * NP), adst.reshape(2 * NP))
    al = _alpha(w, di, den)
    part = _agg(h_cat.reshape(2 * NP, D), gi.reshape(E // B, B),
                dst.reshape(E // B, B), al)
    out = _fin(part, (b_login + b_exec).reshape(1, D))
    return out[:N]


# speedup vs baseline: 47.4146x; 1.0062x over previous
import functools

import jax
import jax.numpy as jnp
from jax import lax
from jax.experimental import pallas as pl
from jax.experimental.pallas import tpu as pltpu
from jax.experimental.pallas import tpu_sc as plsc

N = 10000
E = 320000
D = 128
NP = 10240
NC, NS = 2, 16
NT = NC * NS
EC = E // NT
EB = 2000
NB = EC // EB
B = 125
CH = EC // B
DEN = 2 * NP
DST = DEN // NS
ROWS_PT = NP // NS
CPB = 80

_mesh = plsc.VectorSubcoreMesh(
    core_axis_name="c", subcore_axis_name="s", num_cores=NC, num_subcores=NS)
_sc_params = pltpu.CompilerParams(needs_layout_passes=False)


def _proj_body(x_ref, w_ref, att_ref, h_ref, as_ref, ad_ref):
    for r in range(2):
        h = jnp.dot(x_ref[...], w_ref[r], preferred_element_type=jnp.float32)
        h_ref[r] = h
        as_ref[r] = jnp.sum(h * att_ref[r, 0][None, :], axis=1)
        ad_ref[r] = jnp.sum(h * att_ref[r, 1][None, :], axis=1)


_proj = pl.pallas_call(
    _proj_body,
    out_shape=[
        jax.ShapeDtypeStruct((2, NP, D), jnp.float32),
        jax.ShapeDtypeStruct((2, NP), jnp.float32),
        jax.ShapeDtypeStruct((2, NP), jnp.float32),
    ],
    interpret=False,
)


def _edge_body(src_h, dst_h, tt_h, asrc_h, adst_h,
               w_h, gi_h, di_h, den_h,
               src_v, dst_v, tt_v, as_v, ad_v, w_v, gi_v, di_v,
               den_v, acc_v, tmp_v, den_sh):
    cid = lax.axis_index("c")
    sid = lax.axis_index("s")
    wid = cid * NS + sid
    base = wid * EC

    pltpu.sync_copy(asrc_h, as_v)
    pltpu.sync_copy(adst_h, ad_v)

    zeros16 = jnp.zeros((16,), jnp.float32)

    def zero_body(i, carry):
        den_v[pl.ds(i * 16, 16)] = zeros16
        return carry

    lax.fori_loop(0, DEN // 16, zero_body, 0)

    def block_body(b, carry):
        boff = base + b * EB
        pltpu.sync_copy(src_h.at[pl.ds(boff, EB)], src_v)
        pltpu.sync_copy(dst_h.at[pl.ds(boff, EB)], dst_v)
        pltpu.sync_copy(tt_h.at[pl.ds(boff, EB)], tt_v)

        def edge_body(i, c2):
            off = i * 16
            s = src_v[pl.ds(off, 16)]
            d = dst_v[pl.ds(off, 16)]
            t = tt_v[pl.ds(off, 16)]
            gi = t * NP + s
            di = t * NP + d
            a1 = plsc.load_gather(as_v, [gi])
            a2 = plsc.load_gather(ad_v, [di])
            e = a1 + a2
            e = jnp.where(e >= 0.0, e, e * jnp.float32(0.2))
            wv = jnp.exp(e)
            w_v[pl.ds(off, 16)] = wv
            gi_v[pl.ds(off, 16)] = gi
            di_v[pl.ds(off, 16)] = di
            plsc.addupdate_scatter(den_v, [di], wv)
            return c2

        lax.fori_loop(0, EB // 16, edge_body, 0)
        pltpu.sync_copy(w_v, w_h.at[pl.ds(boff, EB)])
        pltpu.sync_copy(gi_v, gi_h.at[pl.ds(boff, EB)])
        pltpu.sync_copy(di_v, di_h.at[pl.ds(boff, EB)])
        return carry

    lax.fori_loop(0, NB, block_body, 0)

    pltpu.sync_copy(den_v, den_sh.at[sid])
    plsc.subcore_barrier()

    stripe = sid * DST
    pltpu.sync_copy(den_sh.at[0, pl.ds(stripe, DST)], acc_v)
    for k in range(1, NS):
        pltpu.sync_copy(den_sh.at[k, pl.ds(stripe, DST)], tmp_v)

        def add_body(i, carry):
            o = i * 16
            acc_v[pl.ds(o, 16)] = acc_v[pl.ds(o, 16)] + tmp_v[pl.ds(o, 16)]
            return carry

        lax.fori_loop(0, DST // 16, add_body, 0)

    pltpu.sync_copy(acc_v, den_h.at[cid, pl.ds(stripe, DST)])


_edge = functools.partial(
    pl.kernel,
    out_type=[
        jax.ShapeDtypeStruct((E,), jnp.float32),
        jax.ShapeDtypeStruct((E,), jnp.int32),
        jax.ShapeDtypeStruct((E,), jnp.int32),
        jax.ShapeDtypeStruct((NC, DEN), jnp.float32),
    ],
    mesh=_mesh,
    scratch_types=[
        pltpu.VMEM((EB,), jnp.int32),
        pltpu.VMEM((EB,), jnp.int32),
        pltpu.VMEM((EB,), jnp.int32),
        pltpu.VMEM((DEN,), jnp.float32),
        pltpu.VMEM((DEN,), jnp.float32),
        pltpu.VMEM((EB,), jnp.float32),
        pltpu.VMEM((EB,), jnp.int32),
        pltpu.VMEM((EB,), jnp.int32),
        pltpu.VMEM((DEN,), jnp.float32),
        pltpu.VMEM((DST,), jnp.float32),
        pltpu.VMEM((DST,), jnp.float32),
        pltpu.VMEM_SHARED((NS, DEN), jnp.float32),
    ],
    compiler_params=_sc_params,
    interpret=False,
)(_edge_body)


def _alpha_body(w_h, di_h, den_h,
                al_h,
                p0_v, p1_v, rden_v, w_v, di_v,
                rden_sh):
    cid = lax.axis_index("c")
    sid = lax.axis_index("s")
    wid = cid * NS + sid
    base = wid * EC
    stripe = sid * DST

    pltpu.sync_copy(den_h.at[0, pl.ds(stripe, DST)], p0_v)
    pltpu.sync_copy(den_h.at[1, pl.ds(stripe, DST)], p1_v)

    def rden_body(i, carry):
        o = i * 16
        v0 = p0_v[pl.ds(o, 16)]
        v1 = p1_v[pl.ds(o, 16)]
        p0_v[pl.ds(o, 16)] = 1.0 / (v0 + v1 + jnp.float32(1e-16))
        return carry

    lax.fori_loop(0, DST // 16, rden_body, 0)
    pltpu.sync_copy(p0_v, rden_sh.at[pl.ds(stripe, DST)])
    plsc.subcore_barrier()
    pltpu.sync_copy(rden_sh, rden_v)

    def block_body(b, carry):
        boff = base + b * EB
        pltpu.sync_copy(w_h.at[pl.ds(boff, EB)], w_v)
        pltpu.sync_copy(di_h.at[pl.ds(boff, EB)], di_v)

        def al_body(i, c2):
            off = i * 16
            di = di_v[pl.ds(off, 16)]
            rd = plsc.load_gather(rden_v, [di])
            w_v[pl.ds(off, 16)] = w_v[pl.ds(off, 16)] * rd
            return c2

        lax.fori_loop(0, EB // 16, al_body, 0)
        pltpu.sync_copy(w_v, al_h.at[pl.ds(boff, EB)])
        return carry

    lax.fori_loop(0, NB, block_body, 0)


_alpha = functools.partial(
    pl.kernel,
    out_type=jax.ShapeDtypeStruct((E,), jnp.float32),
    mesh=_mesh,
    scratch_types=[
        pltpu.VMEM((DST,), jnp.float32),
        pltpu.VMEM((DST,), jnp.float32),
        pltpu.VMEM((DEN,), jnp.float32),
        pltpu.VMEM((EB,), jnp.float32),
        pltpu.VMEM((EB,), jnp.int32),
        pltpu.VMEM_SHARED((DEN,), jnp.float32),
    ],
    compiler_params=_sc_params,
    interpret=False,
)(_alpha_body)


def _agg_body(hcat_h, gi2_h, dst2_h, al2_h,
              part_h,
              gi_c0, gi_c1, dst_c0, dst_c1, al_c0, al_c1, rows_a, rows_b,
              acc_sh, sem_a, sem_b):
    cid = lax.axis_index("c")
    sid = lax.axis_index("s")
    wid = cid * NS + sid
    cbase = wid * CH

    zeros16 = jnp.zeros((16,), jnp.float32)

    def zrow(i, carry):
        r = i // (D // 16)
        c = (i % (D // 16)) * 16
        rows_a[r, pl.ds(c, 16)] = zeros16
        return carry

    lax.fori_loop(0, CPB * (D // 16), zrow, 0)
    for k in range(ROWS_PT // CPB):
        pltpu.sync_copy(rows_a.at[pl.ds(0, CPB)],
                        acc_sh.at[pl.ds(sid * ROWS_PT + k * CPB, CPB)])
    plsc.subcore_barrier()

    bufs = ((gi_c0, dst_c0, al_c0, rows_a, sem_a),
            (gi_c1, dst_c1, al_c1, rows_b, sem_b))

    def stage_chunk(j, x):
        gi_c, dst_c, al_c, rows, sem = bufs[x]
        pltpu.sync_copy(gi2_h.at[cbase + j], gi_c)
        pltpu.sync_copy(dst2_h.at[cbase + j], dst_c)
        pltpu.sync_copy(al2_h.at[cbase + j], al_c)
        pltpu.async_copy(hcat_h.at[gi_c], rows, sem)

    def process_chunk(x):
        gi_c, dst_c, al_c, rows, sem = bufs[x]
        pltpu.make_async_copy(hcat_h.at[gi_c], rows, sem).wait()

        def edge_mul(q, c2):
            aspl = plsc.load_gather(al_c, [jnp.full((16,), q, jnp.int32)])
            for c in range(D // 16):
                rows[q, pl.ds(c * 16, 16)] = rows[q, pl.ds(c * 16, 16)] * aspl
            return c2

        lax.fori_loop(0, B, edge_mul, 0, unroll=5)
        pltpu.sync_copy(rows, acc_sh.at[dst_c], add=True)

    stage_chunk(0, 0)
    stage_chunk(1, 1)

    def pair_body(k, carry):
        j = k * 2
        process_chunk(0)

        @pl.when(j + 2 < CH)
        def _():
            stage_chunk(j + 2, 0)

        process_chunk(1)

        @pl.when(j + 3 < CH)
        def _():
            stage_chunk(j + 3, 1)

        return carry

    lax.fori_loop(0, CH // 2, pair_body, 0)

    plsc.subcore_barrier()
    for k in range(ROWS_PT // CPB):
        pltpu.sync_copy(acc_sh.at[pl.ds(sid * ROWS_PT + k * CPB, CPB)],
                        rows_a.at[pl.ds(0, CPB)])
        pltpu.sync_copy(rows_a.at[pl.ds(0, CPB)],
                        part_h.at[cid, pl.ds(sid * ROWS_PT + k * CPB, CPB)])


_agg = functools.partial(
    pl.kernel,
    out_type=jax.ShapeDtypeStruct((NC, NP, D), jnp.float32),
    mesh=_mesh,
    scratch_types=[
        pltpu.VMEM((B,), jnp.int32),
        pltpu.VMEM((B,), jnp.int32),
        pltpu.VMEM((B,), jnp.int32),
        pltpu.VMEM((B,), jnp.int32),
        pltpu.VMEM((B,), jnp.float32),
        pltpu.VMEM((B,), jnp.float32),
        pltpu.VMEM((B, D), jnp.float32),
        pltpu.VMEM((B, D), jnp.float32),
        pltpu.VMEM_SHARED((NP, D), jnp.float32),
        pltpu.SemaphoreType.DMA,
        pltpu.SemaphoreType.DMA,
    ],
    compiler_params=_sc_params,
    interpret=False,
)(_agg_body)


def _fin_body(p_ref, b_ref, o_ref):
    o_ref[...] = p_ref[0] + p_ref[1] + b_ref[...]


_fin = pl.pallas_call(
    _fin_body,
    out_shape=jax.ShapeDtypeStruct((NP, D), jnp.float32),
    interpret=False,
)


def kernel(x, edge_index, edge_type, W_login, a_src_login, a_dst_login,
           b_login, W_exec, a_src_exec, a_dst_exec, b_exec):
    x_pad = jnp.pad(x, ((0, NP - N), (0, 0)))
    Ws = jnp.stack([W_login, W_exec])
    atts = jnp.stack([jnp.stack([a_src_login, a_dst_login]),
                      jnp.stack([a_src_exec, a_dst_exec])])
    src = edge_index[0]
    dst = edge_index[1]

    h_cat, asrc, adst = _proj(x_pad, Ws, atts)
    w, gi, di, den = _edge(src, dst, edge_type,
                           asrc.reshape(2 * NP), adst.reshape(2 * NP))
    al = _alpha(w, di, den)
    part = _agg(h_cat.reshape(2 * NP, D), gi.reshape(E // B, B),
                dst.reshape(E // B, B), al.reshape(E // B, B))
    out = _fin(part, (b_login + b_exec).reshape(1, D))
    return out[:N]


# speedup vs baseline: 54.4266x; 1.1479x over previous
import functools

import jax
import jax.numpy as jnp
from jax import lax
from jax.experimental import pallas as pl
from jax.experimental.pallas import tpu as pltpu
from jax.experimental.pallas import tpu_sc as plsc

N = 10000
E = 320000
D = 128
NP = 10240
NC, NS = 2, 16
NT = NC * NS
EC = E // NT
EB = 2000
NB = EC // EB
B = 125
CH = EC // B
DEN = 2 * NP
DST = DEN // NS
ZCH = 80
NZ = N // ZCH

_mesh = plsc.VectorSubcoreMesh(
    core_axis_name="c", subcore_axis_name="s", num_cores=NC, num_subcores=NS)
_sc_params = pltpu.CompilerParams(needs_layout_passes=False)


def _proj_body(x_ref, w_ref, att_ref, h_ref, as_ref, ad_ref):
    for r in range(2):
        h = jnp.dot(x_ref[...], w_ref[r], preferred_element_type=jnp.float32)
        h_ref[r] = h
        as_ref[r] = jnp.sum(h * att_ref[r, 0][None, :], axis=1)
        ad_ref[r] = jnp.sum(h * att_ref[r, 1][None, :], axis=1)


_proj = pl.pallas_call(
    _proj_body,
    out_shape=[
        jax.ShapeDtypeStruct((2, NP, D), jnp.float32),
        jax.ShapeDtypeStruct((2, NP), jnp.float32),
        jax.ShapeDtypeStruct((2, NP), jnp.float32),
    ],
    interpret=False,
)


def _edge_body(src_h, dst_h, tt_h, asrc_h, adst_h,
               w_h, gi_h, di_h, den_h,
               src_v, dst_v, tt_v, as_v, ad_v, w_v, gi_v, di_v,
               den_v, acc_v, tmp_v, den_sh):
    cid = lax.axis_index("c")
    sid = lax.axis_index("s")
    wid = cid * NS + sid
    base = wid * EC

    pltpu.sync_copy(asrc_h, as_v)
    pltpu.sync_copy(adst_h, ad_v)

    zeros16 = jnp.zeros((16,), jnp.float32)

    def zero_body(i, carry):
        den_v[pl.ds(i * 16, 16)] = zeros16
        return carry

    lax.fori_loop(0, DEN // 16, zero_body, 0)

    def block_body(b, carry):
        boff = base + b * EB
        pltpu.sync_copy(src_h.at[pl.ds(boff, EB)], src_v)
        pltpu.sync_copy(dst_h.at[pl.ds(boff, EB)], dst_v)
        pltpu.sync_copy(tt_h.at[pl.ds(boff, EB)], tt_v)

        def edge_body(i, c2):
            off = i * 16
            s = src_v[pl.ds(off, 16)]
            d = dst_v[pl.ds(off, 16)]
            t = tt_v[pl.ds(off, 16)]
            gi = t * NP + s
            di = t * NP + d
            a1 = plsc.load_gather(as_v, [gi])
            a2 = plsc.load_gather(ad_v, [di])
            e = a1 + a2
            e = jnp.where(e >= 0.0, e, e * jnp.float32(0.2))
            wv = jnp.exp(e)
            w_v[pl.ds(off, 16)] = wv
            gi_v[pl.ds(off, 16)] = gi
            di_v[pl.ds(off, 16)] = di
            plsc.addupdate_scatter(den_v, [di], wv)
            return c2

        lax.fori_loop(0, EB // 16, edge_body, 0)
        pltpu.sync_copy(w_v, w_h.at[pl.ds(boff, EB)])
        pltpu.sync_copy(gi_v, gi_h.at[pl.ds(boff, EB)])
        pltpu.sync_copy(di_v, di_h.at[pl.ds(boff, EB)])
        return carry

    lax.fori_loop(0, NB, block_body, 0)

    pltpu.sync_copy(den_v, den_sh.at[sid])
    plsc.subcore_barrier()

    stripe = sid * DST
    pltpu.sync_copy(den_sh.at[0, pl.ds(stripe, DST)], acc_v)
    for k in range(1, NS):
        pltpu.sync_copy(den_sh.at[k, pl.ds(stripe, DST)], tmp_v)

        def add_body(i, carry):
            o = i * 16
            acc_v[pl.ds(o, 16)] = acc_v[pl.ds(o, 16)] + tmp_v[pl.ds(o, 16)]
            return carry

        lax.fori_loop(0, DST // 16, add_body, 0)

    pltpu.sync_copy(acc_v, den_h.at[cid, pl.ds(stripe, DST)])


_edge = functools.partial(
    pl.kernel,
    out_type=[
        jax.ShapeDtypeStruct((E,), jnp.float32),
        jax.ShapeDtypeStruct((E,), jnp.int32),
        jax.ShapeDtypeStruct((E,), jnp.int32),
        jax.ShapeDtypeStruct((NC, DEN), jnp.float32),
    ],
    mesh=_mesh,
    scratch_types=[
        pltpu.VMEM((EB,), jnp.int32),
        pltpu.VMEM((EB,), jnp.int32),
        pltpu.VMEM((EB,), jnp.int32),
        pltpu.VMEM((DEN,), jnp.float32),
        pltpu.VMEM((DEN,), jnp.float32),
        pltpu.VMEM((EB,), jnp.float32),
        pltpu.VMEM((EB,), jnp.int32),
        pltpu.VMEM((EB,), jnp.int32),
        pltpu.VMEM((DEN,), jnp.float32),
        pltpu.VMEM((DST,), jnp.float32),
        pltpu.VMEM((DST,), jnp.float32),
        pltpu.VMEM_SHARED((NS, DEN), jnp.float32),
    ],
    compiler_params=_sc_params,
    interpret=False,
)(_edge_body)


def _alpha_body(w_h, di_h, den_h,
                al_h,
                p0_v, p1_v, rden_v, w_v, di_v,
                rden_sh):
    cid = lax.axis_index("c")
    sid = lax.axis_index("s")
    wid = cid * NS + sid
    base = wid * EC
    stripe = sid * DST

    pltpu.sync_copy(den_h.at[0, pl.ds(stripe, DST)], p0_v)
    pltpu.sync_copy(den_h.at[1, pl.ds(stripe, DST)], p1_v)

    def rden_body(i, carry):
        o = i * 16
        v0 = p0_v[pl.ds(o, 16)]
        v1 = p1_v[pl.ds(o, 16)]
        p0_v[pl.ds(o, 16)] = 1.0 / (v0 + v1 + jnp.float32(1e-16))
        return carry

    lax.fori_loop(0, DST // 16, rden_body, 0)
    pltpu.sync_copy(p0_v, rden_sh.at[pl.ds(stripe, DST)])
    plsc.subcore_barrier()
    pltpu.sync_copy(rden_sh, rden_v)

    def block_body(b, carry):
        boff = base + b * EB
        pltpu.sync_copy(w_h.at[pl.ds(boff, EB)], w_v)
        pltpu.sync_copy(di_h.at[pl.ds(boff, EB)], di_v)

        def al_body(i, c2):
            off = i * 16
            di = di_v[pl.ds(off, 16)]
            rd = plsc.load_gather(rden_v, [di])
            w_v[pl.ds(off, 16)] = w_v[pl.ds(off, 16)] * rd
            return c2

        lax.fori_loop(0, EB // 16, al_body, 0)
        pltpu.sync_copy(w_v, al_h.at[pl.ds(boff, EB)])
        return carry

    lax.fori_loop(0, NB, block_body, 0)


_alpha = functools.partial(
    pl.kernel,
    out_type=jax.ShapeDtypeStruct((E,), jnp.float32),
    mesh=_mesh,
    scratch_types=[
        pltpu.VMEM((DST,), jnp.float32),
        pltpu.VMEM((DST,), jnp.float32),
        pltpu.VMEM((DEN,), jnp.float32),
        pltpu.VMEM((EB,), jnp.float32),
        pltpu.VMEM((EB,), jnp.int32),
        pltpu.VMEM_SHARED((DEN,), jnp.float32),
    ],
    compiler_params=_sc_params,
    interpret=False,
)(_alpha_body)


def _agg_body(hcat_h, gi2_h, dst2_h, al2_h,
              part_h,
              gi_c0, gi_c1, gi_c2, dst_c0, dst_c1, dst_c2,
              al_c0, al_c1, al_c2, rows_0, rows_1, rows_2,
              acc_sh, sem_0, sem_1, sem_2, ssem_0, ssem_1, ssem_2):
    cid = lax.axis_index("c")
    sid = lax.axis_index("s")
    wid = cid * NS + sid
    cbase = wid * CH

    zeros16 = jnp.zeros((16,), jnp.float32)

    def zrow(i, carry):
        r = i // (D // 16)
        c = (i % (D // 16)) * 16
        rows_0[r, pl.ds(c, 16)] = zeros16
        return carry

    lax.fori_loop(0, ZCH * (D // 16), zrow, 0)
    for k in range(-(-NZ // NS)):
        ch = sid + NS * k

        @pl.when(ch < NZ)
        def _():
            pltpu.sync_copy(rows_0.at[pl.ds(0, ZCH)],
                            acc_sh.at[pl.ds(ch * ZCH, ZCH)])

    plsc.subcore_barrier()

    bufs = ((gi_c0, dst_c0, al_c0, rows_0, sem_0, ssem_0),
            (gi_c1, dst_c1, al_c1, rows_1, sem_1, ssem_1),
            (gi_c2, dst_c2, al_c2, rows_2, sem_2, ssem_2))

    def drain_scatter(x):
        gi_c, dst_c, al_c, rows, sem, ssem = bufs[x]
        pltpu.make_async_copy(rows, acc_sh.at[dst_c], ssem).wait()

    def stage_chunk(j, x):
        gi_c, dst_c, al_c, rows, sem, ssem = bufs[x]
        pltpu.sync_copy(gi2_h.at[cbase + j], gi_c)
        pltpu.sync_copy(dst2_h.at[cbase + j], dst_c)
        pltpu.sync_copy(al2_h.at[cbase + j], al_c)
        pltpu.async_copy(hcat_h.at[gi_c], rows, sem)

    def process_chunk(x):
        gi_c, dst_c, al_c, rows, sem, ssem = bufs[x]
        pltpu.make_async_copy(hcat_h.at[gi_c], rows, sem).wait()

        def edge_mul(q, c2):
            aspl = plsc.load_gather(al_c, [jnp.full((16,), q, jnp.int32)])
            for c in range(D // 16):
                rows[q, pl.ds(c * 16, 16)] = rows[q, pl.ds(c * 16, 16)] * aspl
            return c2

        lax.fori_loop(0, B, edge_mul, 0, unroll=5)
        pltpu.async_copy(rows, acc_sh.at[dst_c], ssem, add=True)

    stage_chunk(0, 0)
    stage_chunk(1, 1)

    def rot_body(k, carry):
        j = k * 3
        process_chunk(0)

        @pl.when(k > 0)
        def _():
            drain_scatter(2)
        stage_chunk(j + 2, 2)

        process_chunk(1)
        drain_scatter(0)
        stage_chunk(j + 3, 0)

        process_chunk(2)
        drain_scatter(1)
        stage_chunk(j + 4, 1)

        return carry

    lax.fori_loop(0, (CH - 2) // 3, rot_body, 0)

    process_chunk(0)
    process_chunk(1)
    drain_scatter(2)
    drain_scatter(0)
    drain_scatter(1)

    plsc.subcore_barrier()
    for k in range(-(-NZ // NS)):
        ch = sid + NS * k

        @pl.when(ch < NZ)
        def _():
            pltpu.sync_copy(acc_sh.at[pl.ds(ch * ZCH, ZCH)],
                            rows_0.at[pl.ds(0, ZCH)])
            pltpu.sync_copy(rows_0.at[pl.ds(0, ZCH)],
                            part_h.at[cid, pl.ds(ch * ZCH, ZCH)])


_agg = functools.partial(
    pl.kernel,
    out_type=jax.ShapeDtypeStruct((NC, N, D), jnp.float32),
    mesh=_mesh,
    scratch_types=[
        pltpu.VMEM((B,), jnp.int32),
        pltpu.VMEM((B,), jnp.int32),
        pltpu.VMEM((B,), jnp.int32),
        pltpu.VMEM((B,), jnp.int32),
        pltpu.VMEM((B,), jnp.int32),
        pltpu.VMEM((B,), jnp.int32),
        pltpu.VMEM((B,), jnp.float32),
        pltpu.VMEM((B,), jnp.float32),
        pltpu.VMEM((B,), jnp.float32),
        pltpu.VMEM((B, D), jnp.float32),
        pltpu.VMEM((B, D), jnp.float32),
        pltpu.VMEM((B, D), jnp.float32),
        pltpu.VMEM_SHARED((N, D), jnp.float32),
        pltpu.SemaphoreType.DMA,
        pltpu.SemaphoreType.DMA,
        pltpu.SemaphoreType.DMA,
        pltpu.SemaphoreType.DMA,
        pltpu.SemaphoreType.DMA,
        pltpu.SemaphoreType.DMA,
    ],
    compiler_params=_sc_params,
    interpret=False,
)(_agg_body)


def _fin_body(p_ref, b_ref, o_ref):
    o_ref[...] = p_ref[0] + p_ref[1] + b_ref[...]


_fin = pl.pallas_call(
    _fin_body,
    out_shape=jax.ShapeDtypeStruct((N, D), jnp.float32),
    interpret=False,
)


def kernel(x, edge_index, edge_type, W_login, a_src_login, a_dst_login,
           b_login, W_exec, a_src_exec, a_dst_exec, b_exec):
    x_pad = jnp.pad(x, ((0, NP - N), (0, 0)))
    Ws = jnp.stack([W_login, W_exec])
    atts = jnp.stack([jnp.stack([a_src_login, a_dst_login]),
                      jnp.stack([a_src_exec, a_dst_exec])])
    src = edge_index[0]
    dst = edge_index[1]

    h_cat, asrc, adst = _proj(x_pad, Ws, atts)
    w, gi, di, den = _edge(src, dst, edge_type,
                           asrc.reshape(2 * NP), adst.reshape(2 * NP))
    al = _alpha(w, di, den)
    part = _agg(h_cat.reshape(2 * NP, D), gi.reshape(E // B, B),
                dst.reshape(E // B, B), al.reshape(E // B, B))
    return _fin(part, (b_login + b_exec).reshape(1, D))


# speedup vs baseline: 66.7721x; 1.2268x over previous
import functools

import jax
import jax.numpy as jnp
from jax import lax
from jax.experimental import pallas as pl
from jax.experimental.pallas import tpu as pltpu
from jax.experimental.pallas import tpu_sc as plsc

N = 10000
E = 320000
D = 128
NP = 10240
NC, NS = 2, 16
NT = NC * NS
EC = E // NT
EB = 2000
NB = EC // EB
B = 125
CH = EC // B
DEN = 2 * NP
DST = DEN // NS
ZCH = 80
NZ = N // ZCH

_mesh = plsc.VectorSubcoreMesh(
    core_axis_name="c", subcore_axis_name="s", num_cores=NC, num_subcores=NS)
_sc_params = pltpu.CompilerParams(needs_layout_passes=False)


def _proj_body(x_ref, w_ref, att_ref, h_ref, as_ref, ad_ref):
    for r in range(2):
        h = jnp.dot(x_ref[...], w_ref[r], preferred_element_type=jnp.float32)
        h_ref[r] = h
        as_ref[r] = jnp.sum(h * att_ref[r, 0][None, :], axis=1)
        ad_ref[r] = jnp.sum(h * att_ref[r, 1][None, :], axis=1)


_proj = pl.pallas_call(
    _proj_body,
    out_shape=[
        jax.ShapeDtypeStruct((2, NP, D), jnp.float32),
        jax.ShapeDtypeStruct((2, NP), jnp.float32),
        jax.ShapeDtypeStruct((2, NP), jnp.float32),
    ],
    interpret=False,
)


def _edge_body(src_h, dst_h, tt_h, asrc_h, adst_h,
               w_h, gi_h, di_h, den_h,
               src_v, dst_v, tt_v, as_v, ad_v, w_v, gi_v, di_v,
               den_v, acc_v, tmp_v, den_sh):
    cid = lax.axis_index("c")
    sid = lax.axis_index("s")
    wid = cid * NS + sid
    base = wid * EC

    pltpu.sync_copy(asrc_h, as_v)
    pltpu.sync_copy(adst_h, ad_v)

    zeros16 = jnp.zeros((16,), jnp.float32)

    def zero_body(i, carry):
        den_v[pl.ds(i * 16, 16)] = zeros16
        return carry

    lax.fori_loop(0, DEN // 16, zero_body, 0, unroll=8)

    def block_body(b, carry):
        boff = base + b * EB
        pltpu.sync_copy(src_h.at[pl.ds(boff, EB)], src_v)
        pltpu.sync_copy(dst_h.at[pl.ds(boff, EB)], dst_v)
        pltpu.sync_copy(tt_h.at[pl.ds(boff, EB)], tt_v)

        def edge_body(i, c2):
            off = i * 16
            s = src_v[pl.ds(off, 16)]
            d = dst_v[pl.ds(off, 16)]
            t = tt_v[pl.ds(off, 16)]
            gi = t * NP + s
            di = t * NP + d
            a1 = plsc.load_gather(as_v, [gi])
            a2 = plsc.load_gather(ad_v, [di])
            e = a1 + a2
            e = jnp.where(e >= 0.0, e, e * jnp.float32(0.2))
            wv = jnp.exp(e)
            w_v[pl.ds(off, 16)] = wv
            gi_v[pl.ds(off, 16)] = gi
            di_v[pl.ds(off, 16)] = di
            plsc.addupdate_scatter(den_v, [di], wv)
            return c2

        lax.fori_loop(0, EB // 16, edge_body, 0, unroll=4)
        pltpu.sync_copy(w_v, w_h.at[pl.ds(boff, EB)])
        pltpu.sync_copy(gi_v, gi_h.at[pl.ds(boff, EB)])
        pltpu.sync_copy(di_v, di_h.at[pl.ds(boff, EB)])
        return carry

    lax.fori_loop(0, NB, block_body, 0)

    pltpu.sync_copy(den_v, den_sh.at[sid])
    plsc.subcore_barrier()

    stripe = sid * DST
    pltpu.sync_copy(den_sh.at[0, pl.ds(stripe, DST)], acc_v)
    for k in range(1, NS):
        pltpu.sync_copy(den_sh.at[k, pl.ds(stripe, DST)], tmp_v)

        def add_body(i, carry):
            o = i * 16
            acc_v[pl.ds(o, 16)] = acc_v[pl.ds(o, 16)] + tmp_v[pl.ds(o, 16)]
            return carry

        lax.fori_loop(0, DST // 16, add_body, 0, unroll=8)

    pltpu.sync_copy(acc_v, den_h.at[cid, pl.ds(stripe, DST)])


_edge = functools.partial(
    pl.kernel,
    out_type=[
        jax.ShapeDtypeStruct((E,), jnp.float32),
        jax.ShapeDtypeStruct((E,), jnp.int32),
        jax.ShapeDtypeStruct((E,), jnp.int32),
        jax.ShapeDtypeStruct((NC, DEN), jnp.float32),
    ],
    mesh=_mesh,
    scratch_types=[
        pltpu.VMEM((EB,), jnp.int32),
        pltpu.VMEM((EB,), jnp.int32),
        pltpu.VMEM((EB,), jnp.int32),
        pltpu.VMEM((DEN,), jnp.float32),
        pltpu.VMEM((DEN,), jnp.float32),
        pltpu.VMEM((EB,), jnp.float32),
        pltpu.VMEM((EB,), jnp.int32),
        pltpu.VMEM((EB,), jnp.int32),
        pltpu.VMEM((DEN,), jnp.float32),
        pltpu.VMEM((DST,), jnp.float32),
        pltpu.VMEM((DST,), jnp.float32),
        pltpu.VMEM_SHARED((NS, DEN), jnp.float32),
    ],
    compiler_params=_sc_params,
    interpret=False,
)(_edge_body)


def _alpha_body(w_h, di_h, den_h,
                al_h,
                p0_v, p1_v, rden_v, w_v, di_v,
                rden_sh):
    cid = lax.axis_index("c")
    sid = lax.axis_index("s")
    wid = cid * NS + sid
    base = wid * EC
    stripe = sid * DST

    pltpu.sync_copy(den_h.at[0, pl.ds(stripe, DST)], p0_v)
    pltpu.sync_copy(den_h.at[1, pl.ds(stripe, DST)], p1_v)

    def rden_body(i, carry):
        o = i * 16
        v0 = p0_v[pl.ds(o, 16)]
        v1 = p1_v[pl.ds(o, 16)]
        p0_v[pl.ds(o, 16)] = 1.0 / (v0 + v1 + jnp.float32(1e-16))
        return carry

    lax.fori_loop(0, DST // 16, rden_body, 0, unroll=8)
    pltpu.sync_copy(p0_v, rden_sh.at[pl.ds(stripe, DST)])
    plsc.subcore_barrier()
    pltpu.sync_copy(rden_sh, rden_v)

    def block_body(b, carry):
        boff = base + b * EB
        pltpu.sync_copy(w_h.at[pl.ds(boff, EB)], w_v)
        pltpu.sync_copy(di_h.at[pl.ds(boff, EB)], di_v)

        def al_body(i, c2):
            off = i * 16
            di = di_v[pl.ds(off, 16)]
            rd = plsc.load_gather(rden_v, [di])
            w_v[pl.ds(off, 16)] = w_v[pl.ds(off, 16)] * rd
            return c2

        lax.fori_loop(0, EB // 16, al_body, 0, unroll=8)
        pltpu.sync_copy(w_v, al_h.at[pl.ds(boff, EB)])
        return carry

    lax.fori_loop(0, NB, block_body, 0)


_alpha = functools.partial(
    pl.kernel,
    out_type=jax.ShapeDtypeStruct((E,), jnp.float32),
    mesh=_mesh,
    scratch_types=[
        pltpu.VMEM((DST,), jnp.float32),
        pltpu.VMEM((DST,), jnp.float32),
        pltpu.VMEM((DEN,), jnp.float32),
        pltpu.VMEM((EB,), jnp.float32),
        pltpu.VMEM((EB,), jnp.int32),
        pltpu.VMEM_SHARED((DEN,), jnp.float32),
    ],
    compiler_params=_sc_params,
    interpret=False,
)(_alpha_body)


def _agg_body(hcat_h, pk_h,
              part_h,
              pk_c0, pk_c1, pk_c2, rows_0, rows_1, rows_2,
              acc_sh, sem_0, sem_1, sem_2, ssem_0, ssem_1, ssem_2):
    cid = lax.axis_index("c")
    sid = lax.axis_index("s")
    wid = cid * NS + sid
    cbase = wid * CH

    zeros16 = jnp.zeros((16,), jnp.float32)

    def zrow(i, carry):
        r = i // (D // 16)
        c = (i % (D // 16)) * 16
        rows_0[r, pl.ds(c, 16)] = zeros16
        return carry

    lax.fori_loop(0, ZCH * (D // 16), zrow, 0)
    for k in range(-(-NZ // NS)):
        ch = sid + NS * k

        @pl.when(ch < NZ)
        def _():
            pltpu.sync_copy(rows_0.at[pl.ds(0, ZCH)],
                            acc_sh.at[pl.ds(ch * ZCH, ZCH)])

    plsc.subcore_barrier()

    bufs = ((pk_c0, rows_0, sem_0, ssem_0),
            (pk_c1, rows_1, sem_1, ssem_1),
            (pk_c2, rows_2, sem_2, ssem_2))

    def drain_scatter(x):
        pk_c, rows, sem, ssem = bufs[x]
        pltpu.make_async_copy(rows, acc_sh.at[pk_c.at[1]], ssem).wait()

    def stage_chunk(j, x):
        pk_c, rows, sem, ssem = bufs[x]
        pltpu.sync_copy(pk_h.at[cbase + j], pk_c)
        pltpu.async_copy(hcat_h.at[pk_c.at[0]], rows, sem)

    def process_chunk(x):
        pk_c, rows, sem, ssem = bufs[x]
        pltpu.make_async_copy(hcat_h.at[pk_c.at[0]], rows, sem).wait()

        def edge_mul(q, c2):
            ai = plsc.load_gather(pk_c, [jnp.full((16,), 2, jnp.int32),
                                         jnp.full((16,), q, jnp.int32)])
            aspl = plsc.bitcast(ai, jnp.float32)
            for c in range(D // 16):
                rows[q, pl.ds(c * 16, 16)] = rows[q, pl.ds(c * 16, 16)] * aspl
            return c2

        lax.fori_loop(0, B, edge_mul, 0, unroll=5)
        pltpu.async_copy(rows, acc_sh.at[pk_c.at[1]], ssem, add=True)

    stage_chunk(0, 0)
    stage_chunk(1, 1)

    def rot_body(k, carry):
        j = k * 3
        process_chunk(0)

        @pl.when(k > 0)
        def _():
            drain_scatter(2)
        stage_chunk(j + 2, 2)

        process_chunk(1)
        drain_scatter(0)
        stage_chunk(j + 3, 0)

        process_chunk(2)
        drain_scatter(1)
        stage_chunk(j + 4, 1)

        return carry

    lax.fori_loop(0, (CH - 2) // 3, rot_body, 0)

    process_chunk(0)
    process_chunk(1)
    drain_scatter(2)
    drain_scatter(0)
    drain_scatter(1)

    plsc.subcore_barrier()
    for k in range(-(-NZ // NS)):
        ch = sid + NS * k

        @pl.when(ch < NZ)
        def _():
            pltpu.sync_copy(acc_sh.at[pl.ds(ch * ZCH, ZCH)],
                            rows_0.at[pl.ds(0, ZCH)])
            pltpu.sync_copy(rows_0.at[pl.ds(0, ZCH)],
                            part_h.at[cid, pl.ds(ch * ZCH, ZCH)])


_agg = functools.partial(
    pl.kernel,
    out_type=jax.ShapeDtypeStruct((NC, N, D), jnp.float32),
    mesh=_mesh,
    scratch_types=[
        pltpu.VMEM((3, B), jnp.int32),
        pltpu.VMEM((3, B), jnp.int32),
        pltpu.VMEM((3, B), jnp.int32),
        pltpu.VMEM((B, D), jnp.float32),
        pltpu.VMEM((B, D), jnp.float32),
        pltpu.VMEM((B, D), jnp.float32),
        pltpu.VMEM_SHARED((N, D), jnp.float32),
        pltpu.SemaphoreType.DMA,
        pltpu.SemaphoreType.DMA,
        pltpu.SemaphoreType.DMA,
        pltpu.SemaphoreType.DMA,
        pltpu.SemaphoreType.DMA,
        pltpu.SemaphoreType.DMA,
    ],
    compiler_params=_sc_params,
    interpret=False,
)(_agg_body)


def _fin_body(p_ref, b_ref, o_ref):
    o_ref[...] = p_ref[0] + p_ref[1] + b_ref[...]


_fin = pl.pallas_call(
    _fin_body,
    out_shape=jax.ShapeDtypeStruct((N, D), jnp.float32),
    interpret=False,
)


def kernel(x, edge_index, edge_type, W_login, a_src_login, a_dst_login,
           b_login, W_exec, a_src_exec, a_dst_exec, b_exec):
    x_pad = jnp.pad(x, ((0, NP - N), (0, 0)))
    Ws = jnp.stack([W_login, W_exec])
    atts = jnp.stack([jnp.stack([a_src_login, a_dst_login]),
                      jnp.stack([a_src_exec, a_dst_exec])])
    src = edge_index[0]
    dst = edge_index[1]

    h_cat, asrc, adst = _proj(x_pad, Ws, atts)
    w, gi, di, den = _edge(src, dst, edge_type,
                           asrc.reshape(2 * NP), adst.reshape(2 * NP))
    al = _alpha(w, di, den)
    pk = jnp.stack([gi.reshape(E // B, B), dst.reshape(E // B, B),
                    lax.bitcast_convert_type(al, jnp.int32).reshape(E // B, B)],
                   axis=1)
    part = _agg(h_cat.reshape(2 * NP, D), pk)
    return _fin(part, (b_login + b_exec).reshape(1, D))
